# baseline (speedup 1.0000x reference)
import functools

import jax
import jax.numpy as jnp
import numpy as np
from jax import lax
from jax.experimental import pallas as pl
from jax.experimental.pallas import tpu as pltpu
from jax.experimental.pallas import tpu_sc as plsc

N = 50000
E = 800000
EA = 16
WN = 160
DW = 32

LANES = 128
E_PAD = 802816
N_ROWS = E_PAD // LANES
ROWS_PER_W = N_ROWS // 32

G_ROWS = 14
G_GROUPS = ROWS_PER_W // G_ROWS
S_ROWS = 4
S_GROUPS = ROWS_PER_W // S_ROWS

NP = 50176
TILE_SLICE = NP // 16

TC_B = 2048
QB = TC_B // 4
TC_GRID = E_PAD // TC_B

_INV_SQRT3 = float(1.0 / np.sqrt(3.0))
_INV_SQRT12 = float(1.0 / np.sqrt(12.0))
_INV_SQRT2 = float(1.0 / np.sqrt(2.0))


def _sc_gather(node_p, src_rows):
    mesh = plsc.VectorSubcoreMesh(core_axis_name="c", subcore_axis_name="s")

    @functools.partial(
        pl.kernel,
        out_type=jax.ShapeDtypeStruct((E_PAD, DW), jnp.float32),
        mesh=mesh,
        scratch_types=[
            pltpu.VMEM((G_ROWS, LANES), jnp.int32),
            pltpu.VMEM((G_ROWS * LANES, DW), jnp.float32),
            pltpu.SemaphoreType.DMA,
        ],
        compiler_params=pltpu.CompilerParams(use_tc_tiling_on_sc=False),
    )
    def gather_k(node_hbm, src_hbm, out_hbm, idx_v, gbuf, sem):
        cid = lax.axis_index("c")
        sid = lax.axis_index("s")
        wid = cid * 16 + sid
        row0 = wid * ROWS_PER_W

        def body(g, carry):
            r = row0 + g * G_ROWS
            pltpu.sync_copy(src_hbm.at[pl.ds(r, G_ROWS)], idx_v)
            copies = []
            for j in range(G_ROWS):
                copies.append(
                    pltpu.async_copy(
                        node_hbm.at[idx_v.at[j]],
                        gbuf.at[pl.ds(j * LANES, LANES)],
                        sem,
                    )
                )
            for c in copies:
                c.wait()
            pltpu.sync_copy(gbuf, out_hbm.at[pl.ds(r * LANES, G_ROWS * LANES)])
            return carry

        lax.fori_loop(0, G_GROUPS, body, 0)

    return gather_k(node_p, src_rows)


def _sc_scatter(msg, dst_rows, zeros_hbm):
    mesh = plsc.VectorSubcoreMesh(core_axis_name="c", subcore_axis_name="s")

    @functools.partial(
        pl.kernel,
        out_type=jax.ShapeDtypeStruct((2, NP, DW), jnp.float32),
        mesh=mesh,
        scratch_types=[
            pltpu.VMEM((S_ROWS, LANES), jnp.int32),
            pltpu.VMEM((S_ROWS * LANES, DW), jnp.float32),
            pltpu.VMEM_SHARED((NP, DW), jnp.float32),
            pltpu.SemaphoreType.DMA,
        ],
        compiler_params=pltpu.CompilerParams(use_tc_tiling_on_sc=False),
    )
    def scatter_k(msg_hbm, dst_hbm, zero_hbm, out_hbm, idx_v, mbuf, acc, sem):
        cid = lax.axis_index("c")
        sid = lax.axis_index("s")
        wid = cid * 16 + sid
        row0 = wid * ROWS_PER_W

        sl = pl.ds(sid * TILE_SLICE, TILE_SLICE)
        pltpu.sync_copy(zero_hbm.at[sl], acc.at[sl])
        plsc.subcore_barrier()

        def body(g, carry):
            r = row0 + g * S_ROWS
            pltpu.sync_copy(dst_hbm.at[pl.ds(r, S_ROWS)], idx_v)
            pltpu.async_copy(
                msg_hbm.at[pl.ds(r * LANES, S_ROWS * LANES)], mbuf, sem
            ).wait()
            for j in range(S_ROWS):
                pltpu.sync_copy(
                    mbuf.at[pl.ds(j * LANES, LANES)],
                    acc.at[idx_v.at[j]],
                    add=True,
                )
            return carry

        lax.fori_loop(0, S_GROUPS, body, 0)
        plsc.subcore_barrier()
        pltpu.sync_copy(acc.at[sl], out_hbm.at[cid, sl])

    return scatter_k(msg, dst_rows, zeros_hbm)


def _tc_body(ea_ref, sh_ref, g_ref, w1t_ref, b1_ref, w2t_ref, b2_ref, out_ref):
    ea = ea_ref[...]
    h = jnp.dot(w1t_ref[...], ea, preferred_element_type=jnp.float32) + b1_ref[...]
    h = h * jax.nn.sigmoid(h)
    w = jnp.dot(w2t_ref[...], h, preferred_element_type=jnp.float32) + b2_ref[...]

    gt = g_ref[...].T
    shf = sh_ref[...]

    pieces = []
    for r in range(4):
        lo = r * QB
        wr = w[:, lo: lo + QB]
        sh = shf[:, lo: lo + QB]
        base = r * DW
        x0 = gt[base: base + 8]
        x1m = [gt[base + 8 + 4 * m: base + 12 + 4 * m] for m in range(3)]
        s0 = sh[0:1]
        s1 = [sh[1 + m: 2 + m] for m in range(3)]

        x0s0 = x0 * s0
        out0 = wr[0:8] * x0s0[0:1]
        for u in range(1, 8):
            out0 = out0 + wr[8 * u: 8 * u + 8] * x0s0[u: u + 1]
        dot = (x1m[0] * s1[0] + x1m[1] * s1[1] + x1m[2] * s1[2]) * _INV_SQRT3
        for u in range(4):
            out0 = out0 + wr[112 + 8 * u: 120 + 8 * u] * dot[u: u + 1]
        out0 = out0 * _INV_SQRT12

        t2 = wr[64:68] * x0[0:1]
        for u in range(1, 8):
            t2 = t2 + wr[64 + 4 * u: 68 + 4 * u] * x0[u: u + 1]

        o1 = []
        for m in range(3):
            mp1, mp2 = (m + 1) % 3, (m + 2) % 3
            cross = (x1m[mp1] * s1[mp2] - x1m[mp2] * s1[mp1]) * _INV_SQRT2
            t3 = wr[96:100] * x1m[m][0:1]
            t5 = wr[144:148] * cross[0:1]
            for u in range(1, 4):
                t3 = t3 + wr[96 + 4 * u: 100 + 4 * u] * x1m[m][u: u + 1]
                t5 = t5 + wr[144 + 4 * u: 148 + 4 * u] * cross[u: u + 1]
            o1.append((t2 * s1[m] + t3 * s0 + t5) * 0.25)

        zpad = out0[0:4] * 0.0
        pieces.extend([out0] + o1 + [zpad, zpad, zpad])

    out_ref[...] = jnp.concatenate(pieces, axis=0).T


def _tc_compute(ea_t, sh_t, g2, W1t, b1c, W2t, b2c):
    return pl.pallas_call(
        _tc_body,
        grid=(TC_GRID,),
        in_specs=[
            pl.BlockSpec((EA, TC_B), lambda i: (0, i)),
            pl.BlockSpec((4, TC_B), lambda i: (0, i)),
            pl.BlockSpec((QB, LANES), lambda i: (i, 0)),
            pl.BlockSpec((EA, EA), lambda i: (0, 0)),
            pl.BlockSpec((EA, 1), lambda i: (0, 0)),
            pl.BlockSpec((WN, EA), lambda i: (0, 0)),
            pl.BlockSpec((WN, 1), lambda i: (0, 0)),
        ],
        out_specs=pl.BlockSpec((QB, LANES), lambda i: (i, 0)),
        out_shape=jax.ShapeDtypeStruct((E_PAD // 4, LANES), jnp.float32),
    )(ea_t, sh_t, g2, W1t, b1c, W2t, b2c)


def _tc_add(partials):
    def body(a_ref, o_ref):
        o_ref[...] = a_ref[0] + a_ref[1]

    return pl.pallas_call(
        body,
        grid=(NP // 3136,),
        in_specs=[pl.BlockSpec((2, 3136, DW), lambda i: (0, i, 0))],
        out_specs=pl.BlockSpec((3136, DW), lambda i: (i, 0)),
        out_shape=jax.ShapeDtypeStruct((NP, DW), jnp.float32),
    )(partials)


def kernel(node_attr, edge_index, edge_attr, edge_sh, W1, b1, W2, b2):
    node_p = jnp.concatenate(
        [node_attr[:, 0:8]]
        + [node_attr[:, 8 + u * 3 + m: 9 + u * 3 + m] for m in range(3) for u in range(4)]
        + [jnp.zeros((N, DW - 20), jnp.float32)],
        axis=1,
    )
    pad = E_PAD - E
    spread_src = (jnp.arange(pad, dtype=jnp.int32) * 61) % N
    spread_dst = N + (jnp.arange(pad, dtype=jnp.int32) % (NP - N))
    src_rows = jnp.concatenate([edge_index[0], spread_src]).reshape(N_ROWS, LANES)
    dst_rows = jnp.concatenate([edge_index[1], spread_dst]).reshape(N_ROWS, LANES)

    ea_pad = jnp.concatenate([edge_attr, jnp.zeros((pad, EA), jnp.float32)])
    sh_pad = jnp.concatenate([edge_sh, jnp.zeros((pad, 4), jnp.float32)])
    ea_t = ea_pad.reshape(TC_GRID, QB, 4, EA).transpose(3, 0, 2, 1).reshape(EA, E_PAD)
    sh_t = sh_pad.reshape(TC_GRID, QB, 4, 4).transpose(3, 0, 2, 1).reshape(4, E_PAD)

    gathered = _sc_gather(node_p, src_rows)
    g2 = gathered.reshape(E_PAD // 4, LANES)
    msg2 = _tc_compute(
        ea_t, sh_t, g2, W1.T, b1.reshape(EA, 1), W2.T, b2.reshape(WN, 1)
    )
    msg = msg2.reshape(E_PAD, DW)
    partials = _sc_scatter(msg, dst_rows, jnp.zeros((NP, DW), jnp.float32))
    out_mm = _tc_add(partials)
    return jnp.concatenate(
        [out_mm[:N, 0:8]]
        + [out_mm[:N, 8 + m * 4 + u: 9 + m * 4 + u] for u in range(4) for m in range(3)],
        axis=1,
    )


# speedup vs baseline: 1.4323x; 1.4323x over previous
import functools

import jax
import jax.numpy as jnp
import numpy as np
from jax import lax
from jax.experimental import pallas as pl
from jax.experimental.pallas import tpu as pltpu
from jax.experimental.pallas import tpu_sc as plsc

N = 50000
E = 800000
EA = 16
WN = 160
DW = 32

LANES = 128
E_PAD = 802816
N_ROWS = E_PAD // LANES
ROWS_PER_W = N_ROWS // 32

G_ROWS = 14
G_GROUPS = ROWS_PER_W // G_ROWS
S_ROWS = 4
S_GROUPS = ROWS_PER_W // S_ROWS

NP = 50176
TILE_SLICE = NP // 16

TC_B = 2048
QB = TC_B // 4
TC_GRID = E_PAD // TC_B

_INV_SQRT3 = float(1.0 / np.sqrt(3.0))
_INV_SQRT12 = float(1.0 / np.sqrt(12.0))
_INV_SQRT2 = float(1.0 / np.sqrt(2.0))


def _sc_gather(node_p, src_rows):
    mesh = plsc.VectorSubcoreMesh(core_axis_name="c", subcore_axis_name="s")

    @functools.partial(
        pl.kernel,
        out_type=jax.ShapeDtypeStruct((E_PAD, DW), jnp.float32),
        mesh=mesh,
        scratch_types=[
            pltpu.VMEM((G_ROWS, LANES), jnp.int32),
            pltpu.VMEM((G_ROWS * LANES, DW), jnp.float32),
            pltpu.SemaphoreType.DMA,
        ],
        compiler_params=pltpu.CompilerParams(use_tc_tiling_on_sc=False),
    )
    def gather_k(node_hbm, src_hbm, out_hbm, idx_v, gbuf, sem):
        cid = lax.axis_index("c")
        sid = lax.axis_index("s")
        wid = cid * 16 + sid
        row0 = wid * ROWS_PER_W

        def body(g, carry):
            r = row0 + g * G_ROWS
            pltpu.sync_copy(src_hbm.at[pl.ds(r, G_ROWS)], idx_v)
            copies = []
            for j in range(G_ROWS):
                copies.append(
                    pltpu.async_copy(
                        node_hbm.at[idx_v.at[j]],
                        gbuf.at[pl.ds(j * LANES, LANES)],
                        sem,
                    )
                )
            for c in copies:
                c.wait()
            pltpu.sync_copy(gbuf, out_hbm.at[pl.ds(r * LANES, G_ROWS * LANES)])
            return carry

        lax.fori_loop(0, G_GROUPS, body, 0)

    return gather_k(node_p, src_rows)


def _sc_scatter(msg, dst_rows, zeros_hbm):
    mesh = plsc.VectorSubcoreMesh(core_axis_name="c", subcore_axis_name="s")

    @functools.partial(
        pl.kernel,
        out_type=jax.ShapeDtypeStruct((2, NP, DW), jnp.float32),
        mesh=mesh,
        scratch_types=[
            pltpu.VMEM((S_ROWS, LANES), jnp.int32),
            pltpu.VMEM((S_ROWS * LANES, DW), jnp.float32),
            pltpu.VMEM_SHARED((NP, DW), jnp.float32),
            pltpu.SemaphoreType.DMA,
        ],
        compiler_params=pltpu.CompilerParams(use_tc_tiling_on_sc=False),
    )
    def scatter_k(msg_hbm, dst_hbm, zero_hbm, out_hbm, idx_v, mbuf, acc, sem):
        cid = lax.axis_index("c")
        sid = lax.axis_index("s")
        wid = cid * 16 + sid
        row0 = wid * ROWS_PER_W

        sl = pl.ds(sid * TILE_SLICE, TILE_SLICE)
        pltpu.sync_copy(zero_hbm.at[sl], acc.at[sl])
        plsc.subcore_barrier()

        def body(g, carry):
            r = row0 + g * S_ROWS
            pltpu.sync_copy(dst_hbm.at[pl.ds(r, S_ROWS)], idx_v)
            pltpu.async_copy(
                msg_hbm.at[pl.ds(r * LANES, S_ROWS * LANES)], mbuf, sem
            ).wait()
            for j in range(S_ROWS):
                pltpu.sync_copy(
                    mbuf.at[pl.ds(j * LANES, LANES)],
                    acc.at[idx_v.at[j]],
                    add=True,
                )
            return carry

        lax.fori_loop(0, S_GROUPS, body, 0)
        plsc.subcore_barrier()
        pltpu.sync_copy(acc.at[sl], out_hbm.at[cid, sl])

    return scatter_k(msg, dst_rows, zeros_hbm)


def _tc_body(eash_ref, g_ref, w1t_ref, b1_ref, w2t_ref, b2_ref, out_ref):
    et = eash_ref[...].T
    gt = g_ref[...].T

    pieces = []
    for r in range(4):
        base = r * DW
        ea = et[base: base + EA]
        sh = et[base + EA: base + EA + 4]
        h = jnp.dot(w1t_ref[...], ea, preferred_element_type=jnp.float32) + b1_ref[...]
        h = h * jax.nn.sigmoid(h)
        wr = jnp.dot(w2t_ref[...], h, preferred_element_type=jnp.float32) + b2_ref[...]
        x0 = gt[base: base + 8]
        x1m = [gt[base + 8 + 4 * m: base + 12 + 4 * m] for m in range(3)]
        s0 = sh[0:1]
        s1 = [sh[1 + m: 2 + m] for m in range(3)]

        x0s0 = x0 * s0
        out0 = wr[0:8] * x0s0[0:1]
        for u in range(1, 8):
            out0 = out0 + wr[8 * u: 8 * u + 8] * x0s0[u: u + 1]
        dot = (x1m[0] * s1[0] + x1m[1] * s1[1] + x1m[2] * s1[2]) * _INV_SQRT3
        for u in range(4):
            out0 = out0 + wr[112 + 8 * u: 120 + 8 * u] * dot[u: u + 1]
        out0 = out0 * _INV_SQRT12

        t2 = wr[64:68] * x0[0:1]
        for u in range(1, 8):
            t2 = t2 + wr[64 + 4 * u: 68 + 4 * u] * x0[u: u + 1]

        o1 = []
        for m in range(3):
            mp1, mp2 = (m + 1) % 3, (m + 2) % 3
            cross = (x1m[mp1] * s1[mp2] - x1m[mp2] * s1[mp1]) * _INV_SQRT2
            t3 = wr[96:100] * x1m[m][0:1]
            t5 = wr[144:148] * cross[0:1]
            for u in range(1, 4):
                t3 = t3 + wr[96 + 4 * u: 100 + 4 * u] * x1m[m][u: u + 1]
                t5 = t5 + wr[144 + 4 * u: 148 + 4 * u] * cross[u: u + 1]
            o1.append((t2 * s1[m] + t3 * s0 + t5) * 0.25)

        zpad = out0[0:4] * 0.0
        pieces.extend([out0] + o1 + [zpad, zpad, zpad])

    out_ref[...] = jnp.concatenate(pieces, axis=0).T


def _tc_compute(eash2, g2, W1t, b1c, W2t, b2c):
    return pl.pallas_call(
        _tc_body,
        grid=(TC_GRID,),
        in_specs=[
            pl.BlockSpec((QB, LANES), lambda i: (i, 0)),
            pl.BlockSpec((QB, LANES), lambda i: (i, 0)),
            pl.BlockSpec((EA, EA), lambda i: (0, 0)),
            pl.BlockSpec((EA, 1), lambda i: (0, 0)),
            pl.BlockSpec((WN, EA), lambda i: (0, 0)),
            pl.BlockSpec((WN, 1), lambda i: (0, 0)),
        ],
        out_specs=pl.BlockSpec((QB, LANES), lambda i: (i, 0)),
        out_shape=jax.ShapeDtypeStruct((E_PAD // 4, LANES), jnp.float32),
    )(eash2, g2, W1t, b1c, W2t, b2c)


def _tc_add(partials):
    def body(a_ref, o_ref):
        o_ref[...] = a_ref[0] + a_ref[1]

    return pl.pallas_call(
        body,
        grid=(NP // 3136,),
        in_specs=[pl.BlockSpec((2, 3136, DW), lambda i: (0, i, 0))],
        out_specs=pl.BlockSpec((3136, DW), lambda i: (i, 0)),
        out_shape=jax.ShapeDtypeStruct((NP, DW), jnp.float32),
    )(partials)


def kernel(node_attr, edge_index, edge_attr, edge_sh, W1, b1, W2, b2):
    node_p = jnp.concatenate(
        [node_attr[:, 0:8]]
        + [node_attr[:, 8 + u * 3 + m: 9 + u * 3 + m] for m in range(3) for u in range(4)]
        + [jnp.zeros((N, DW - 20), jnp.float32)],
        axis=1,
    )
    pad = E_PAD - E
    spread_src = (jnp.arange(pad, dtype=jnp.int32) * 61) % N
    spread_dst = N + (jnp.arange(pad, dtype=jnp.int32) % (NP - N))
    src_rows = jnp.concatenate([edge_index[0], spread_src]).reshape(N_ROWS, LANES)
    dst_rows = jnp.concatenate([edge_index[1], spread_dst]).reshape(N_ROWS, LANES)

    ea_pad = jnp.concatenate([edge_attr, jnp.zeros((pad, EA), jnp.float32)])
    sh_pad = jnp.concatenate([edge_sh, jnp.zeros((pad, 4), jnp.float32)])
    eash = jnp.concatenate(
        [ea_pad, sh_pad, jnp.zeros((E_PAD, DW - EA - 4), jnp.float32)], axis=1
    )
    eash2 = eash.reshape(E_PAD // 4, LANES)

    gathered = _sc_gather(node_p, src_rows)
    g2 = gathered.reshape(E_PAD // 4, LANES)
    msg2 = _tc_compute(
        eash2, g2, W1.T, b1.reshape(EA, 1), W2.T, b2.reshape(WN, 1)
    )
    msg = msg2.reshape(E_PAD, DW)
    partials = _sc_scatter(msg, dst_rows, jnp.zeros((NP, DW), jnp.float32))
    out_mm = _tc_add(partials)
    return jnp.concatenate(
        [out_mm[:N, 0:8]]
        + [out_mm[:N, 8 + m * 4 + u: 9 + m * 4 + u] for u in range(4) for m in range(3)],
        axis=1,
    )


# speedup vs baseline: 1.6929x; 1.1820x over previous
import functools

import jax
import jax.numpy as jnp
import numpy as np
from jax import lax
from jax.experimental import pallas as pl
from jax.experimental.pallas import tpu as pltpu
from jax.experimental.pallas import tpu_sc as plsc

N = 50000
E = 800000
EA = 16
WN = 160
DW = 32

LANES = 128
E_PAD = 802816
N_ROWS = E_PAD // LANES
ROWS_PER_W = N_ROWS // 32

G_ROWS = 14
G_GROUPS = ROWS_PER_W // G_ROWS
S_ROWS = 4
S_GROUPS = ROWS_PER_W // S_ROWS

NP = 50176
TILE_SLICE = NP // 16

TC_B = 4096
QB = TC_B // 4
TC_GRID = E_PAD // TC_B

_INV_SQRT3 = float(1.0 / np.sqrt(3.0))
_INV_SQRT12 = float(1.0 / np.sqrt(12.0))
_INV_SQRT2 = float(1.0 / np.sqrt(2.0))


def _sc_gather(node_p, src_rows):
    mesh = plsc.VectorSubcoreMesh(core_axis_name="c", subcore_axis_name="s")

    @functools.partial(
        pl.kernel,
        out_type=jax.ShapeDtypeStruct((E_PAD, DW), jnp.float32),
        mesh=mesh,
        scratch_types=[
            pltpu.VMEM((G_ROWS, LANES), jnp.int32),
            pltpu.VMEM((G_ROWS * LANES, DW), jnp.float32),
            pltpu.SemaphoreType.DMA,
        ],
        compiler_params=pltpu.CompilerParams(use_tc_tiling_on_sc=False),
    )
    def gather_k(node_hbm, src_hbm, out_hbm, idx_v, gbuf, sem):
        cid = lax.axis_index("c")
        sid = lax.axis_index("s")
        wid = cid * 16 + sid
        row0 = wid * ROWS_PER_W

        def body(g, carry):
            r = row0 + g * G_ROWS
            pltpu.sync_copy(src_hbm.at[pl.ds(r, G_ROWS)], idx_v)
            copies = []
            for j in range(G_ROWS):
                copies.append(
                    pltpu.async_copy(
                        node_hbm.at[idx_v.at[j]],
                        gbuf.at[pl.ds(j * LANES, LANES)],
                        sem,
                    )
                )
            for c in copies:
                c.wait()
            pltpu.sync_copy(gbuf, out_hbm.at[pl.ds(r * LANES, G_ROWS * LANES)])
            return carry

        lax.fori_loop(0, G_GROUPS, body, 0)

    return gather_k(node_p, src_rows)


def _sc_scatter(msg, dst_rows, zeros_hbm):
    mesh = plsc.VectorSubcoreMesh(core_axis_name="c", subcore_axis_name="s")

    @functools.partial(
        pl.kernel,
        out_type=jax.ShapeDtypeStruct((2, NP, DW), jnp.float32),
        mesh=mesh,
        scratch_types=[
            pltpu.VMEM((S_ROWS, LANES), jnp.int32),
            pltpu.VMEM((S_ROWS * LANES, DW), jnp.float32),
            pltpu.VMEM_SHARED((NP, DW), jnp.float32),
            pltpu.SemaphoreType.DMA,
        ],
        compiler_params=pltpu.CompilerParams(use_tc_tiling_on_sc=False),
    )
    def scatter_k(msg_hbm, dst_hbm, zero_hbm, out_hbm, idx_v, mbuf, acc, sem):
        cid = lax.axis_index("c")
        sid = lax.axis_index("s")
        wid = cid * 16 + sid
        row0 = wid * ROWS_PER_W

        sl = pl.ds(sid * TILE_SLICE, TILE_SLICE)
        pltpu.sync_copy(zero_hbm.at[sl], acc.at[sl])
        plsc.subcore_barrier()

        def body(g, carry):
            r = row0 + g * S_ROWS
            pltpu.sync_copy(dst_hbm.at[pl.ds(r, S_ROWS)], idx_v)
            pltpu.async_copy(
                msg_hbm.at[pl.ds(r * LANES, S_ROWS * LANES)], mbuf, sem
            ).wait()
            for j in range(S_ROWS):
                pltpu.sync_copy(
                    mbuf.at[pl.ds(j * LANES, LANES)],
                    acc.at[idx_v.at[j]],
                    add=True,
                )
            return carry

        lax.fori_loop(0, S_GROUPS, body, 0)
        plsc.subcore_barrier()
        pltpu.sync_copy(acc.at[sl], out_hbm.at[cid, sl])

    return scatter_k(msg, dst_rows, zeros_hbm)


def _tc_body(ea_ref, sh_ref, g_ref, w1t_ref, b1_ref, w2t_ref, b2_ref, out_ref):
    ea = ea_ref[...]
    h = jnp.dot(w1t_ref[...], ea, preferred_element_type=jnp.float32) + b1_ref[...]
    h = h * jax.nn.sigmoid(h)
    w = jnp.dot(w2t_ref[...], h, preferred_element_type=jnp.float32) + b2_ref[...]
    shf = sh_ref[...]
    gt = g_ref[...].T

    pieces = []
    for r in range(4):
        base = r * DW
        lo = r * QB
        wr = w[:, lo: lo + QB]
        sh = shf[:, lo: lo + QB]
        x0 = gt[base: base + 8]
        x1m = [gt[base + 8 + 4 * m: base + 12 + 4 * m] for m in range(3)]
        s0 = sh[0:1]
        s1 = [sh[1 + m: 2 + m] for m in range(3)]

        x0s0 = x0 * s0
        out0 = wr[0:8] * x0s0[0:1]
        for u in range(1, 8):
            out0 = out0 + wr[8 * u: 8 * u + 8] * x0s0[u: u + 1]
        dot = (x1m[0] * s1[0] + x1m[1] * s1[1] + x1m[2] * s1[2]) * _INV_SQRT3
        for u in range(4):
            out0 = out0 + wr[112 + 8 * u: 120 + 8 * u] * dot[u: u + 1]
        out0 = out0 * _INV_SQRT12

        t2 = wr[64:68] * x0[0:1]
        for u in range(1, 8):
            t2 = t2 + wr[64 + 4 * u: 68 + 4 * u] * x0[u: u + 1]

        o1 = []
        for m in range(3):
            mp1, mp2 = (m + 1) % 3, (m + 2) % 3
            cross = (x1m[mp1] * s1[mp2] - x1m[mp2] * s1[mp1]) * _INV_SQRT2
            t3 = wr[96:100] * x1m[m][0:1]
            t5 = wr[144:148] * cross[0:1]
            for u in range(1, 4):
                t3 = t3 + wr[96 + 4 * u: 100 + 4 * u] * x1m[m][u: u + 1]
                t5 = t5 + wr[144 + 4 * u: 148 + 4 * u] * cross[u: u + 1]
            o1.append((t2 * s1[m] + t3 * s0 + t5) * 0.25)

        zpad = out0[0:4] * 0.0
        pieces.extend([out0] + o1 + [zpad, zpad, zpad])

    out_ref[...] = jnp.concatenate(pieces, axis=0).T


def _tc_compute(ea_t, sh_t, g2, W1t, b1c, W2t, b2c):
    return pl.pallas_call(
        _tc_body,
        grid=(TC_GRID,),
        in_specs=[
            pl.BlockSpec((EA, TC_B), lambda i: (0, i)),
            pl.BlockSpec((4, TC_B), lambda i: (0, i)),
            pl.BlockSpec((QB, LANES), lambda i: (i, 0)),
            pl.BlockSpec((EA, EA), lambda i: (0, 0)),
            pl.BlockSpec((EA, 1), lambda i: (0, 0)),
            pl.BlockSpec((WN, EA), lambda i: (0, 0)),
            pl.BlockSpec((WN, 1), lambda i: (0, 0)),
        ],
        out_specs=pl.BlockSpec((QB, LANES), lambda i: (i, 0)),
        out_shape=jax.ShapeDtypeStruct((E_PAD // 4, LANES), jnp.float32),
    )(ea_t, sh_t, g2, W1t, b1c, W2t, b2c)


def _tc_add(partials):
    def body(a_ref, o_ref):
        o_ref[...] = a_ref[0] + a_ref[1]

    return pl.pallas_call(
        body,
        grid=(NP // 3136,),
        in_specs=[pl.BlockSpec((2, 3136, DW), lambda i: (0, i, 0))],
        out_specs=pl.BlockSpec((3136, DW), lambda i: (i, 0)),
        out_shape=jax.ShapeDtypeStruct((NP, DW), jnp.float32),
    )(partials)


def kernel(node_attr, edge_index, edge_attr, edge_sh, W1, b1, W2, b2):
    node_p = jnp.concatenate(
        [node_attr[:, 0:8]]
        + [node_attr[:, 8 + u * 3 + m: 9 + u * 3 + m] for m in range(3) for u in range(4)]
        + [jnp.zeros((N, DW - 20), jnp.float32)],
        axis=1,
    )
    pad = E_PAD - E
    spread_src = (jnp.arange(pad, dtype=jnp.int32) * 61) % N
    spread_dst = N + (jnp.arange(pad, dtype=jnp.int32) % (NP - N))
    src_st = jnp.concatenate([edge_index[0], spread_src])         .reshape(TC_GRID, 4, QB).transpose(0, 2, 1).reshape(N_ROWS, LANES)
    dst_st = jnp.concatenate([edge_index[1], spread_dst])         .reshape(TC_GRID, 4, QB).transpose(0, 2, 1).reshape(N_ROWS, LANES)

    ea_t = jnp.pad(edge_attr.T, ((0, 0), (0, pad)))
    sh_t = jnp.pad(edge_sh.T, ((0, 0), (0, pad)))

    gathered = _sc_gather(node_p, src_st)
    g2 = gathered.reshape(E_PAD // 4, LANES)
    msg2 = _tc_compute(
        ea_t, sh_t, g2, W1.T, b1.reshape(EA, 1), W2.T, b2.reshape(WN, 1)
    )
    msg = msg2.reshape(E_PAD, DW)
    partials = _sc_scatter(msg, dst_st, jnp.zeros((NP, DW), jnp.float32))
    out_mm = _tc_add(partials)
    return jnp.concatenate(
        [out_mm[:N, 0:8]]
        + [out_mm[:N, 8 + m * 4 + u: 9 + m * 4 + u] for u in range(4) for m in range(3)],
        axis=1,
    )


# speedup vs baseline: 2.3792x; 1.4054x over previous
import functools

import jax
import jax.numpy as jnp
import numpy as np
from jax import lax
from jax.experimental import pallas as pl
from jax.experimental.pallas import tpu as pltpu
from jax.experimental.pallas import tpu_sc as plsc

N = 50000
E = 800000
EA = 16
WN = 160
DW = 32

LANES = 128
E_PAD = 802816
N_ROWS = E_PAD // LANES
ROWS_PER_W = N_ROWS // 32

G_ROWS = 14
G_GROUPS = ROWS_PER_W // G_ROWS
S_ROWS = 4
S_GROUPS = ROWS_PER_W // S_ROWS

NP = 50176
TILE_SLICE = NP // 16

TC_B = 4096
QB = TC_B // 4
TC_GRID = E_PAD // TC_B

_P128 = np.zeros((128, 128), np.float32)
for _blk in range(4):
    for _j in range(20):
        _cin = _j if _j < 8 else 8 + ((_j - 8) % 3) * 4 + (_j - 8) // 3
        _P128[_blk * 32 + _cin, _blk * 32 + _j] = 1.0

_INV_SQRT3 = float(1.0 / np.sqrt(3.0))
_INV_SQRT12 = float(1.0 / np.sqrt(12.0))
_INV_SQRT2 = float(1.0 / np.sqrt(2.0))


def _sc_gather(node_p, src_rows):
    mesh = plsc.VectorSubcoreMesh(core_axis_name="c", subcore_axis_name="s")

    @functools.partial(
        pl.kernel,
        out_type=jax.ShapeDtypeStruct((E_PAD, DW), jnp.float32),
        mesh=mesh,
        scratch_types=[
            pltpu.VMEM((G_ROWS, LANES), jnp.int32),
            pltpu.VMEM((G_ROWS * LANES, DW), jnp.float32),
            pltpu.SemaphoreType.DMA,
        ],
        compiler_params=pltpu.CompilerParams(use_tc_tiling_on_sc=False),
    )
    def gather_k(node_hbm, src_hbm, out_hbm, idx_v, gbuf, sem):
        cid = lax.axis_index("c")
        sid = lax.axis_index("s")
        wid = cid * 16 + sid
        row0 = wid * ROWS_PER_W

        def body(g, carry):
            r = row0 + g * G_ROWS
            pltpu.sync_copy(src_hbm.at[pl.ds(r, G_ROWS)], idx_v)
            copies = []
            for j in range(G_ROWS):
                copies.append(
                    pltpu.async_copy(
                        node_hbm.at[idx_v.at[j]],
                        gbuf.at[pl.ds(j * LANES, LANES)],
                        sem,
                    )
                )
            for c in copies:
                c.wait()
            pltpu.sync_copy(gbuf, out_hbm.at[pl.ds(r * LANES, G_ROWS * LANES)])
            return carry

        lax.fori_loop(0, G_GROUPS, body, 0)

    return gather_k(node_p, src_rows)


def _sc_scatter(msg, dst_rows, zeros_hbm):
    mesh = plsc.VectorSubcoreMesh(core_axis_name="c", subcore_axis_name="s")

    @functools.partial(
        pl.kernel,
        out_type=jax.ShapeDtypeStruct((2, NP, DW), jnp.float32),
        mesh=mesh,
        scratch_types=[
            pltpu.VMEM((S_ROWS, LANES), jnp.int32),
            pltpu.VMEM((S_ROWS * LANES, DW), jnp.float32),
            pltpu.VMEM_SHARED((NP, DW), jnp.float32),
            pltpu.SemaphoreType.DMA,
        ],
        compiler_params=pltpu.CompilerParams(use_tc_tiling_on_sc=False),
    )
    def scatter_k(msg_hbm, dst_hbm, zero_hbm, out_hbm, idx_v, mbuf, acc, sem):
        cid = lax.axis_index("c")
        sid = lax.axis_index("s")
        wid = cid * 16 + sid
        row0 = wid * ROWS_PER_W

        sl = pl.ds(sid * TILE_SLICE, TILE_SLICE)
        pltpu.sync_copy(zero_hbm.at[sl], acc.at[sl])
        plsc.subcore_barrier()

        def body(g, carry):
            r = row0 + g * S_ROWS
            pltpu.sync_copy(dst_hbm.at[pl.ds(r, S_ROWS)], idx_v)
            pltpu.async_copy(
                msg_hbm.at[pl.ds(r * LANES, S_ROWS * LANES)], mbuf, sem
            ).wait()
            for j in range(S_ROWS):
                pltpu.sync_copy(
                    mbuf.at[pl.ds(j * LANES, LANES)],
                    acc.at[idx_v.at[j]],
                    add=True,
                )
            return carry

        lax.fori_loop(0, S_GROUPS, body, 0)
        plsc.subcore_barrier()
        pltpu.sync_copy(acc.at[sl], out_hbm.at[cid, sl])

    return scatter_k(msg, dst_rows, zeros_hbm)


def _tc_body(ea_ref, sh_ref, g_ref, w1t_ref, b1_ref, w2t_ref, b2_ref, out_ref):
    ea = ea_ref[...]
    h = jnp.dot(w1t_ref[...], ea, preferred_element_type=jnp.float32) + b1_ref[...]
    h = h * jax.nn.sigmoid(h)
    w = jnp.dot(w2t_ref[...], h, preferred_element_type=jnp.float32) + b2_ref[...]
    shf = sh_ref[...]
    gt = g_ref[...].T

    pieces = []
    for r in range(4):
        base = r * DW
        lo = r * QB
        wr = w[:, lo: lo + QB]
        sh = shf[:, lo: lo + QB]
        x0 = gt[base: base + 8]
        x1m = [gt[base + 8 + 4 * m: base + 12 + 4 * m] for m in range(3)]
        s0 = sh[0:1]
        s1 = [sh[1 + m: 2 + m] for m in range(3)]

        x0s0 = x0 * s0
        out0 = wr[0:8] * x0s0[0:1]
        for u in range(1, 8):
            out0 = out0 + wr[8 * u: 8 * u + 8] * x0s0[u: u + 1]
        dot = (x1m[0] * s1[0] + x1m[1] * s1[1] + x1m[2] * s1[2]) * _INV_SQRT3
        for u in range(4):
            out0 = out0 + wr[112 + 8 * u: 120 + 8 * u] * dot[u: u + 1]
        out0 = out0 * _INV_SQRT12

        t2 = wr[64:68] * x0[0:1]
        for u in range(1, 8):
            t2 = t2 + wr[64 + 4 * u: 68 + 4 * u] * x0[u: u + 1]

        o1 = []
        for m in range(3):
            mp1, mp2 = (m + 1) % 3, (m + 2) % 3
            cross = (x1m[mp1] * s1[mp2] - x1m[mp2] * s1[mp1]) * _INV_SQRT2
            t3 = wr[96:100] * x1m[m][0:1]
            t5 = wr[144:148] * cross[0:1]
            for u in range(1, 4):
                t3 = t3 + wr[96 + 4 * u: 100 + 4 * u] * x1m[m][u: u + 1]
                t5 = t5 + wr[144 + 4 * u: 148 + 4 * u] * cross[u: u + 1]
            o1.append((t2 * s1[m] + t3 * s0 + t5) * 0.25)

        zpad = out0[0:4] * 0.0
        pieces.extend([out0] + o1 + [zpad, zpad, zpad])

    out_ref[...] = jnp.concatenate(pieces, axis=0).T


def _tc_compute(ea_t, sh_t, g2, W1t, b1c, W2t, b2c):
    return pl.pallas_call(
        _tc_body,
        grid=(TC_GRID,),
        in_specs=[
            pl.BlockSpec((EA, TC_B), lambda i: (0, i)),
            pl.BlockSpec((4, TC_B), lambda i: (0, i)),
            pl.BlockSpec((QB, LANES), lambda i: (i, 0)),
            pl.BlockSpec((EA, EA), lambda i: (0, 0)),
            pl.BlockSpec((EA, 1), lambda i: (0, 0)),
            pl.BlockSpec((WN, EA), lambda i: (0, 0)),
            pl.BlockSpec((WN, 1), lambda i: (0, 0)),
        ],
        out_specs=pl.BlockSpec((QB, LANES), lambda i: (i, 0)),
        out_shape=jax.ShapeDtypeStruct((E_PAD // 4, LANES), jnp.float32),
    )(ea_t, sh_t, g2, W1t, b1c, W2t, b2c)


def _tc_add(partials_packed, p128):
    def body(a_ref, p_ref, o_ref):
        o_ref[...] = jnp.dot(
            a_ref[0] + a_ref[1], p_ref[...], preferred_element_type=jnp.float32
        )

    rows = NP // 4
    blk = rows // 8
    return pl.pallas_call(
        body,
        grid=(8,),
        in_specs=[
            pl.BlockSpec((2, blk, LANES), lambda i: (0, i, 0)),
            pl.BlockSpec((LANES, LANES), lambda i: (0, 0)),
        ],
        out_specs=pl.BlockSpec((blk, LANES), lambda i: (i, 0)),
        out_shape=jax.ShapeDtypeStruct((rows, LANES), jnp.float32),
    )(partials_packed, p128)


def kernel(node_attr, edge_index, edge_attr, edge_sh, W1, b1, W2, b2):
    node_t = node_attr.T
    node_p = jnp.concatenate(
        [node_t[0:8]]
        + [node_t[8 + u * 3 + m: 9 + u * 3 + m] for m in range(3) for u in range(4)]
        + [jnp.zeros((DW - 20, N), jnp.float32)],
        axis=0,
    ).T
    pad = E_PAD - E
    spread_src = (jnp.arange(pad, dtype=jnp.int32) * 61) % N
    spread_dst = N + (jnp.arange(pad, dtype=jnp.int32) % (NP - N))
    src_st = jnp.concatenate([edge_index[0], spread_src])         .reshape(TC_GRID, 4, QB).transpose(0, 2, 1).reshape(N_ROWS, LANES)
    dst_st = jnp.concatenate([edge_index[1], spread_dst])         .reshape(TC_GRID, 4, QB).transpose(0, 2, 1).reshape(N_ROWS, LANES)

    ea_t = jnp.pad(edge_attr.T, ((0, 0), (0, pad)))
    sh_t = jnp.pad(edge_sh.T, ((0, 0), (0, pad)))

    gathered = _sc_gather(node_p, src_st)
    g2 = gathered.reshape(E_PAD // 4, LANES)
    msg2 = _tc_compute(
        ea_t, sh_t, g2, W1.T, b1.reshape(EA, 1), W2.T, b2.reshape(WN, 1)
    )
    msg = msg2.reshape(E_PAD, DW)
    partials = _sc_scatter(msg, dst_st, jnp.zeros((NP, DW), jnp.float32))
    outp = _tc_add(partials.reshape(2, NP // 4, LANES), jnp.asarray(_P128))
    return outp.reshape(NP, DW)[:N, :20]
